# baseline (speedup 1.0000x reference)
import functools

import jax
import jax.numpy as jnp
from jax import lax
from jax.experimental import pallas as pl
from jax.experimental.pallas import tpu as pltpu
from jax.experimental.pallas import tpu_sc as plsc

_VOCAB_PER_FIELD = 38461
_N_FIELDS = 26
_BATCH = 16384
_D = 16
_B = _BATCH * _N_FIELDS
_NW = 32
_BPW = _B // _NW
_CHB = 64
_CHUNK = _CHB * _N_FIELDS
_NCHUNK = _BPW // _CHUNK
_NBUF = 2
_SUPER = 208
_NSUPER = _BPW // _SUPER

_mesh = plsc.VectorSubcoreMesh(core_axis_name="c", subcore_axis_name="s")


@functools.partial(
    pl.kernel,
    mesh=_mesh,
    out_type=jax.ShapeDtypeStruct((_N_FIELDS, _D, _BATCH), jnp.float32),
    compiler_params=pltpu.CompilerParams(use_tc_tiling_on_sc=False, needs_layout_passes=False),
    scratch_types=[
        pltpu.VMEM((_BPW,), jnp.int32),
        pltpu.VMEM((_NBUF, _CHUNK, _D), jnp.float32),
        pltpu.VMEM((_NBUF, _N_FIELDS, _D, _CHB), jnp.float32),
        pltpu.VMEM((16,), jnp.int32),
        pltpu.SemaphoreType.DMA,
        pltpu.SemaphoreType.DMA,
        pltpu.SemaphoreType.DMA,
        pltpu.SemaphoreType.DMA,
    ],
)
def _embedding_gather(x_hbm, table_hbm, out_hbm, idx_v, rows_v, stage_v,
                      ridx_v, gsem0, gsem1, ssem0, ssem1):
    gsems = [gsem0, gsem1]
    ssems = [ssem0, ssem1]
    wid = lax.axis_index("s") * 2 + lax.axis_index("c")
    base = wid * _BPW
    batch_base = wid * (_BPW // _N_FIELDS)

    pltpu.sync_copy(x_hbm.at[pl.ds(base, _BPW)], idx_v)

    lane = lax.iota(jnp.int32, 16)
    offs = [((k * 16 + lane) % _N_FIELDS) * _VOCAB_PER_FIELD for k in range(13)]

    def fixup(g, carry):
        s0 = g * _SUPER
        for k in range(13):
            s = s0 + k * 16
            idx_v[pl.ds(s, 16)] = idx_v[pl.ds(s, 16)] + offs[k]
        return carry

    lax.fori_loop(0, _NSUPER, fixup, 0)

    def issue_gather(j):
        b = j % _NBUF
        return pltpu.async_copy(
            table_hbm.at[idx_v.at[pl.ds(j * _CHUNK, _CHUNK)]], rows_v.at[b],
            gsems[b])

    def issue_store(j):
        b = j % _NBUF
        return pltpu.async_copy(
            stage_v.at[b],
            out_hbm.at[:, :, pl.ds(batch_base + j * _CHB, _CHB)],
            ssems[b])

    row_lane = lane * _N_FIELDS
    one_v = jnp.full((16,), 1, jnp.int32)
    half_vs = [jnp.full((16,), h * 16 * _N_FIELDS, jnp.int32)
               for h in range(1, _CHB // 16)]
    dcols = [jnp.full((16,), d, jnp.int32) for d in range(_D)]

    def transpose_chunk(b):
        rows = rows_v.at[b]
        stage = stage_v.at[b]
        ridx_v[...] = row_lane

        def field(f, carry):
            ridx = ridx_v[...]
            for h in range(_CHB // 16):
                rh = ridx if h == 0 else ridx + half_vs[h - 1]
                for d in range(_D):
                    stage[f, d, pl.ds(h * 16, 16)] = plsc.load_gather(
                        rows, [rh, dcols[d]])
            ridx_v[...] = ridx + one_v
            return carry

        lax.fori_loop(0, _N_FIELDS, field, 0)

    stores = [None] * _NCHUNK
    pending = issue_gather(0)
    for j in range(_NCHUNK):
        pending.wait()
        if j + 1 < _NCHUNK:
            pending = issue_gather(j + 1)
        if j >= _NBUF:
            stores[j - _NBUF].wait()
        transpose_chunk(j % _NBUF)
        stores[j] = issue_store(j)
    for j in range(_NCHUNK - _NBUF, _NCHUNK):
        stores[j].wait()


def kernel(x, table):
    planes = _embedding_gather(x.reshape(-1).astype(jnp.int32), table)
    return jnp.transpose(planes, (2, 0, 1))


# speedup vs baseline: 1.4413x; 1.4413x over previous
import functools

import jax
import jax.numpy as jnp
from jax import lax
from jax.experimental import pallas as pl
from jax.experimental.pallas import tpu as pltpu
from jax.experimental.pallas import tpu_sc as plsc

_VOCAB_PER_FIELD = 38461
_N_FIELDS = 26
_BATCH = 16384
_D = 16
_NROW = 999986
_SLAB = 1015808
_NPLANE = _N_FIELDS * _D
_NW = 32
_PPW = _NPLANE // _NW
_NBUF = 2

_mesh = plsc.VectorSubcoreMesh(core_axis_name="c", subcore_axis_name="s")


_RCH = 16384
_NJ = 62


def _detile_body(in_ref, out_hbm, sem):
    i = pl.program_id(0)
    j = pl.program_id(1)
    copies = [pltpu.make_async_copy(
        in_ref.at[dd],
        out_hbm.at[pl.ds((i * 8 + dd) * _SLAB + j * _RCH, _RCH)], sem)
        for dd in range(8)]
    for c in copies:
        c.start()
    for c in copies:
        c.wait()


_tc_detile = pl.pallas_call(
    _detile_body,
    grid=(2, _NJ),
    in_specs=[pl.BlockSpec((8, _RCH), lambda i, j: (i, j))],
    out_specs=pl.BlockSpec(memory_space=pltpu.HBM),
    scratch_shapes=[pltpu.SemaphoreType.DMA],
    out_shape=jax.ShapeDtypeStruct((_D * _SLAB,), jnp.float32),
)


@functools.partial(
    pl.kernel,
    mesh=_mesh,
    out_type=jax.ShapeDtypeStruct((_N_FIELDS, _D, _BATCH), jnp.float32),
    compiler_params=pltpu.CompilerParams(
        use_tc_tiling_on_sc=False, needs_layout_passes=False),
    scratch_types=[
        pltpu.VMEM((_NBUF, _BATCH), jnp.int32),
        pltpu.VMEM((_NBUF, _BATCH), jnp.float32),
        pltpu.SemaphoreType.DMA,
        pltpu.SemaphoreType.DMA,
        pltpu.SemaphoreType.DMA,
        pltpu.SemaphoreType.DMA,
    ],
)
def _plane_gather(xt_hbm, tbl_hbm, out_hbm, idx_v, val_v,
                  gsem0, gsem1, ssem0, ssem1):
    gsems = [gsem0, gsem1]
    ssems = [ssem0, ssem1]
    wid = lax.axis_index("s") * 2 + lax.axis_index("c")
    p0 = wid * _PPW

    def plane_fd(k):
        p = p0 + k
        return p // _D, p % _D

    def stage_idx(k):
        b = k % _NBUF
        f, d = plane_fd(k)
        pltpu.sync_copy(xt_hbm.at[pl.ds(f * _BATCH, _BATCH)], idx_v.at[b])
        addend = jnp.full((16,), f * _VOCAB_PER_FIELD + d * _SLAB, jnp.int32)

        def fix(g, carry):
            s = g * 16
            idx_v[b, pl.ds(s, 16)] = idx_v[b, pl.ds(s, 16)] + addend
            return carry

        lax.fori_loop(0, _BATCH // 16, fix, 0)

    def issue_gather(k):
        b = k % _NBUF
        return pltpu.async_copy(tbl_hbm.at[idx_v.at[b]], val_v.at[b],
                                gsems[b])

    def issue_store(k):
        b = k % _NBUF
        f, d = plane_fd(k)
        return pltpu.async_copy(val_v.at[b], out_hbm.at[f, d], ssems[b])

    stores = [None] * _PPW
    stage_idx(0)
    pending = issue_gather(0)
    for k in range(_PPW):
        if k + 1 < _PPW:
            if k + 1 >= _NBUF:
                stores[k + 1 - _NBUF].wait()
            stage_idx(k + 1)
        pending.wait()
        if k + 1 < _PPW:
            pending = issue_gather(k + 1)
        stores[k] = issue_store(k)
    for k in range(max(0, _PPW - _NBUF), _PPW):
        stores[k].wait()


def kernel(x, table):
    tbl_dmajor = _tc_detile(jnp.transpose(table, (1, 0)))
    xt = jnp.transpose(x, (1, 0)).reshape(-1).astype(jnp.int32)
    planes = _plane_gather(xt, tbl_dmajor)
    return jnp.transpose(planes, (2, 0, 1))
